# baseline (speedup 1.0000x reference)
import dataclasses

import jax
import jax.numpy as jnp
from jax import lax
from jax.experimental import pallas as pl
from jax.experimental.pallas import tpu as pltpu
from jax.experimental.pallas import tpu_sc as plsc

_L = 8192
_D = 768
_NW = 32
_RPW = _L // _NW
_W = 32
_NCH = _RPW // _W
_LANES = 16
_NTB = 3
_NPB = 2


def _sc_body(tok_hbm, pos_hbm, ids_hbm, spec_hbm, o_hbm,
             idx_v, spec_v,
             tb0, tb1, tb2, pb0, pb1,
             ts0, ts1, ts2, ps0, ps1, ws0, ws1, ws2):
    tbufs = (tb0, tb1, tb2)
    pbufs = (pb0, pb1)
    tsems = (ts0, ts1, ts2)
    psems = (ps0, ps1)
    wsems = (ws0, ws1, ws2)

    c_id = lax.axis_index("c")
    s_id = lax.axis_index("s")
    wid = s_id * 2 + c_id
    is_w0 = wid == 0
    base = wid * _RPW

    pltpu.sync_copy(ids_hbm.at[wid], idx_v)
    pltpu.sync_copy(spec_hbm, spec_v)

    @pl.when(is_w0)
    def _patch():
        lane = lax.iota(jnp.int32, 16)
        src = jnp.maximum(lane, 1)
        v = plsc.load_gather(idx_v, [jnp.zeros((16,), jnp.int32), src])
        idx_v[0, pl.ds(0, _LANES)] = v

    def start_tok(c):
        return pltpu.async_copy(tok_hbm.at[idx_v.at[c]], tbufs[c % _NTB],
                                tsems[c % _NTB])

    def start_pos(c):
        pb = pbufs[c % _NPB]
        sem = psems[c % _NPB]
        if c == 0:
            @pl.when(is_w0)
            def _():
                pltpu.async_copy(pos_hbm.at[pl.ds(0, _D)], pb.at[pl.ds(0, _D)],
                                 sem)
                pltpu.async_copy(pos_hbm.at[pl.ds(0, (_W - 1) * _D)],
                                 pb.at[pl.ds(_D, (_W - 1) * _D)], sem)

            @pl.when(jnp.logical_not(is_w0))
            def _():
                pltpu.async_copy(pos_hbm.at[pl.ds((base - 1) * _D, _W * _D)],
                                 pb, sem)
        else:
            off = (base + c * _W - 1) * _D
            pltpu.async_copy(pos_hbm.at[pl.ds(off, _W * _D)], pb, sem)

    def wait_pos(c):
        pltpu.make_async_copy(pos_hbm.at[pl.ds(0, _W * _D)], pbufs[c % _NPB],
                              psems[c % _NPB]).wait()

    tok_cp = {}
    writes = {}
    tok_cp[0] = start_tok(0)
    start_pos(0)
    tok_cp[1] = start_tok(1)
    start_pos(1)

    for c in range(_NCH):
        b = c % _NTB
        tok_v = tbufs[b]
        pos_v = pbufs[c % _NPB]
        tok_cp[c].wait()
        wait_pos(c)

        @pl.loop(0, _W, step=2)
        def _row(r):
            for dr in range(2):
                for col in range(0, _D, _LANES):
                    plsc.addupdate(tok_v.at[r + dr, pl.ds(col, _LANES)],
                                   pos_v[pl.ds((r + dr) * _D + col, _LANES)])

        if c == 0:
            @pl.when(is_w0)
            def _special():
                for col in range(0, _D, _LANES):
                    tok_v[1, pl.ds(col, _LANES)] = spec_v[pl.ds(col, _LANES)]

        writes[c] = pltpu.async_copy(
            tok_v, o_hbm.at[0, pl.ds(base + c * _W, _W)], wsems[b])

        nxt = c + 2
        if nxt < _NCH:
            if nxt - _NTB >= 0:
                writes[nxt - _NTB].wait()
            tok_cp[nxt] = start_tok(nxt)
            start_pos(nxt)

    for c in range(_NCH - _NTB, _NCH):
        writes[c].wait()


@jax.jit
def _embed(token_embedding, position_embedding, ids, spec):
    mesh = plsc.VectorSubcoreMesh(core_axis_name="c", subcore_axis_name="s")
    cp = pltpu.CompilerParams()
    if "needs_layout_passes" in pltpu.CompilerParams.__dataclass_fields__:
        cp = dataclasses.replace(cp, needs_layout_passes=False)
    run = pl.kernel(
        _sc_body,
        compiler_params=cp,
        out_type=jax.ShapeDtypeStruct((1, _L, _D), jnp.float32),
        mesh=mesh,
        scratch_types=(
            [
                pltpu.VMEM((_NCH, _W), jnp.int32),
                pltpu.VMEM((_D,), jnp.float32),
            ]
            + [pltpu.VMEM((_W, _D), jnp.float32)] * _NTB
            + [pltpu.VMEM((_W * _D,), jnp.float32)] * _NPB
            + [pltpu.SemaphoreType.DMA] * 8
        ),
    )
    return run(token_embedding, position_embedding, ids, spec)


def kernel(input_ids, token_embedding, position_embedding, special_token_embedding):
    ids = input_ids.reshape(_NW, _NCH, _W)
    spec = special_token_embedding.reshape(_D)
    pos_flat = position_embedding.reshape(-1)
    return _embed(token_embedding, pos_flat, ids, spec)


# speedup vs baseline: 1.0562x; 1.0562x over previous
import jax
import jax.numpy as jnp
from jax import lax
from jax.experimental import pallas as pl
from jax.experimental.pallas import tpu as pltpu
from jax.experimental.pallas import tpu_sc as plsc

_L = 8192
_D = 768
_NW = 32
_RPW = _L // _NW
_W = 16
_NCH = _RPW // _W
_LANES = 16
_NTB = 3
_NPB = 2
_PW = _W + 8


def _sc_body(tok_hbm, pos_hbm, tokidx_hbm, spec_hbm, o_hbm,
             idx_v, spec_v,
             tb0, tb1, tb2, pb0, pb1,
             ts0, ts1, ts2, ps0, ps1, ws0, ws1, ws2):
    tbufs = (tb0, tb1, tb2)
    pbufs = (pb0, pb1)
    tsems = (ts0, ts1, ts2)
    psems = (ps0, ps1)
    wsems = (ws0, ws1, ws2)

    c_id = lax.axis_index("c")
    s_id = lax.axis_index("s")
    wid = s_id * 2 + c_id
    is_w0 = wid == 0
    base = wid * _RPW

    pltpu.sync_copy(tokidx_hbm.at[wid], idx_v)
    pltpu.sync_copy(spec_hbm, spec_v)

    def start_tok(c):
        return pltpu.async_copy(tok_hbm.at[idx_v.at[c]], tbufs[c % _NTB],
                                tsems[c % _NTB])

    def start_pos(c):
        pb = pbufs[c % _NPB]
        sem = psems[c % _NPB]
        start = base + c * _W - 8
        if c == 0:
            start = jnp.maximum(start, 0)
        start = pl.multiple_of(start, 8)
        return pltpu.async_copy(pos_hbm.at[pl.ds(start, _PW)], pb, sem)

    tok_cp = {}
    pos_cp = {}
    writes = {}
    tok_cp[0] = start_tok(0)
    pos_cp[0] = start_pos(0)
    tok_cp[1] = start_tok(1)
    pos_cp[1] = start_pos(1)

    for c in range(_NCH):
        b = c % _NTB
        tok_v = tbufs[b]
        pos_v = pbufs[c % _NPB]
        tok_cp[c].wait()
        pos_cp[c].wait()

        def add_rows(lo, hi, shift):
            @pl.loop(lo, hi, step=2)
            def _row(r):
                for dr in range(2):
                    for col in range(0, _D, _LANES):
                        tok_v[r + dr, pl.ds(col, _LANES)] += (
                            pos_v[r + dr + shift, pl.ds(col, _LANES)])

        if c == 0:
            @pl.when(is_w0)
            def _w0():
                for col in range(0, _D, _LANES):
                    tok_v[0, pl.ds(col, _LANES)] += pos_v[0, pl.ds(col, _LANES)]
                    tok_v[1, pl.ds(col, _LANES)] = spec_v[pl.ds(col, _LANES)]
                add_rows(2, _W, -1)

            @pl.when(jnp.logical_not(is_w0))
            def _rest():
                add_rows(0, _W, 7)
        else:
            add_rows(0, _W, 7)

        writes[c] = pltpu.async_copy(
            tok_v, o_hbm.at[pl.ds(base + c * _W, _W)], wsems[b])

        nxt = c + 2
        if nxt < _NCH:
            if nxt - _NTB >= 0:
                writes[nxt - _NTB].wait()
            tok_cp[nxt] = start_tok(nxt)
            pos_cp[nxt] = start_pos(nxt)

    for c in range(_NCH - _NTB, _NCH):
        writes[c].wait()


@jax.jit
def _embed(token_embedding, position_embedding, tok_idx, spec):
    mesh = plsc.VectorSubcoreMesh(core_axis_name="c", subcore_axis_name="s")
    run = pl.kernel(
        _sc_body,
        out_type=jax.ShapeDtypeStruct((_L, _D), jnp.float32),
        mesh=mesh,
        scratch_types=(
            [
                pltpu.VMEM((_NCH, _W), jnp.int32),
                pltpu.VMEM((_D,), jnp.float32),
            ]
            + [pltpu.VMEM((_W, _D), jnp.float32)] * _NTB
            + [pltpu.VMEM((_PW, _D), jnp.float32)] * _NPB
            + [pltpu.SemaphoreType.DMA] * 8
        ),
    )
    return run(token_embedding, position_embedding, tok_idx, spec)


def kernel(input_ids, token_embedding, position_embedding, special_token_embedding):
    ids = input_ids[0]
    tok_idx = jnp.concatenate(
        [ids[1:2], jnp.zeros((1,), jnp.int32), ids[2:]]
    ).reshape(_NW, _NCH, _W)
    spec = special_token_embedding.reshape(_D)
    out = _embed(token_embedding, position_embedding, tok_idx, spec)
    return out[None]


# speedup vs baseline: 1.7057x; 1.6149x over previous
import jax
import jax.numpy as jnp
from jax import lax
from jax.experimental import pallas as pl
from jax.experimental.pallas import tpu as pltpu
from jax.experimental.pallas import tpu_sc as plsc

_L = 8192
_D = 768
_NW = 32
_RPW = _L // _NW
_W = 32
_NCH = _RPW // _W
_LANES = 16
_NTB = 3
_NPB = 2


def _sc_body(tok_hbm, pos_hbm, tokidx_hbm, posidx_hbm, spec_hbm, o_hbm,
             idx_v, pidx_v, spec_v,
             tb0, tb1, tb2, pb0, pb1,
             ts0, ts1, ts2, ps0, ps1, ws0, ws1, ws2):
    tbufs = (tb0, tb1, tb2)
    pbufs = (pb0, pb1)
    tsems = (ts0, ts1, ts2)
    psems = (ps0, ps1)
    wsems = (ws0, ws1, ws2)

    c_id = lax.axis_index("c")
    s_id = lax.axis_index("s")
    wid = s_id * 2 + c_id
    base = wid * _RPW

    pltpu.sync_copy(tokidx_hbm.at[wid], idx_v)
    pltpu.sync_copy(posidx_hbm.at[wid], pidx_v)
    pltpu.sync_copy(spec_hbm, spec_v)

    def start_tok(c):
        return pltpu.async_copy(tok_hbm.at[idx_v.at[c]], tbufs[c % _NTB],
                                tsems[c % _NTB])

    def start_pos(c):
        return pltpu.async_copy(pos_hbm.at[pidx_v.at[c]], pbufs[c % _NPB],
                                psems[c % _NPB])

    tok_cp = {}
    pos_cp = {}
    writes = {}
    tok_cp[0] = start_tok(0)
    pos_cp[0] = start_pos(0)
    tok_cp[1] = start_tok(1)
    pos_cp[1] = start_pos(1)

    for c in range(_NCH):
        b = c % _NTB
        tok_v = tbufs[b]
        pos_v = pbufs[c % _NPB]
        tok_cp[c].wait()
        pos_cp[c].wait()

        @pl.loop(0, _W, step=2)
        def _row(r):
            for dr in range(2):
                for col in range(0, _D, _LANES):
                    tok_v[r + dr, pl.ds(col, _LANES)] += (
                        pos_v[r + dr, pl.ds(col, _LANES)])

        if c == 0:
            @pl.when(wid == 0)
            def _special():
                for col in range(0, _D, _LANES):
                    tok_v[1, pl.ds(col, _LANES)] = spec_v[pl.ds(col, _LANES)]

        writes[c] = pltpu.async_copy(
            tok_v, o_hbm.at[0, pl.ds(base + c * _W, _W)], wsems[b])

        nxt = c + 2
        if nxt < _NCH:
            if nxt - _NTB >= 0:
                writes[nxt - _NTB].wait()
            tok_cp[nxt] = start_tok(nxt)
            pos_cp[nxt] = start_pos(nxt)

    for c in range(_NCH - _NTB, _NCH):
        writes[c].wait()


@jax.jit
def _embed(token_embedding, position_embedding, tok_idx, pos_idx, spec):
    mesh = plsc.VectorSubcoreMesh(core_axis_name="c", subcore_axis_name="s")
    run = pl.kernel(
        _sc_body,
        out_type=jax.ShapeDtypeStruct((1, _L, _D), jnp.float32),
        mesh=mesh,
        scratch_types=(
            [
                pltpu.VMEM((_NCH, _W), jnp.int32),
                pltpu.VMEM((_NCH, _W), jnp.int32),
                pltpu.VMEM((_D,), jnp.float32),
            ]
            + [pltpu.VMEM((_W, _D), jnp.float32)] * (_NTB + _NPB)
            + [pltpu.SemaphoreType.DMA] * 8
        ),
    )
    return run(token_embedding, position_embedding, tok_idx, pos_idx, spec)


def kernel(input_ids, token_embedding, position_embedding, special_token_embedding):
    ids = input_ids[0]
    tok_idx = jnp.concatenate(
        [ids[1:2], jnp.zeros((1,), jnp.int32), ids[2:]]
    ).reshape(_NW, _NCH, _W)
    j = jnp.arange(_L, dtype=jnp.int32)
    pos_idx = jnp.maximum(j - 1, 0).reshape(_NW, _NCH, _W)
    spec = special_token_embedding.reshape(_D)
    return _embed(token_embedding, position_embedding, tok_idx, pos_idx, spec)
